# Initial kernel scaffold; baseline (speedup 1.0000x reference)
import jax
import jax.numpy as jnp
from jax.experimental import pallas as pl


def kernel(x, adj, W1, b1, Wx, bx, W2, b2):
    raise NotImplementedError("write your pallas kernel here")


# speedup vs baseline: 13.0105x; 13.0105x over previous
import functools

import jax
import jax.numpy as jnp
from jax import lax
from jax.experimental import pallas as pl
from jax.experimental.pallas import tpu as pltpu
from jax.experimental.pallas import tpu_sc as plsc

N = 10000
E = 320000
NFEAT = 128
NHID = 128
NCLASS = 64

CHUNK = 128
NCHUNKS = E // CHUNK
NSC = 2
NTILES = 16
NW = NSC * NTILES
NP = 10240
RPT = NP // NTILES
DEG_W = 128


@functools.cache
def _mesh():
    return plsc.VectorSubcoreMesh(core_axis_name="c", subcore_axis_name="s")


@functools.cache
def _deg_kernel_fn():
    @functools.partial(
        pl.kernel,
        out_type=jax.ShapeDtypeStruct((NSC * NP, DEG_W), jnp.float32),
        mesh=_mesh(),
        scratch_types=[
            pltpu.VMEM((CHUNK,), jnp.int32),
            pltpu.VMEM((CHUNK, DEG_W), jnp.float32),
            pltpu.VMEM_SHARED((NP, DEG_W), jnp.float32),
        ],
    )
    def _deg_kernel(dst_hbm, ones_hbm, zeros_hbm, out_hbm, didx, ones_v, acc):
        c = lax.axis_index("c")
        s = lax.axis_index("s")
        wid = s * NSC + c
        r0 = s * RPT
        pltpu.sync_copy(ones_hbm, ones_v)
        pltpu.sync_copy(zeros_hbm.at[pl.ds(r0, RPT)], acc.at[pl.ds(r0, RPT)])
        plsc.subcore_barrier()

        n_iter = (jnp.int32(NCHUNKS // NW)
                  + (wid < NCHUNKS % NW).astype(jnp.int32))

        def body(i, carry):
            j = wid + NW * i
            pltpu.sync_copy(dst_hbm.at[j], didx)
            pltpu.sync_copy(ones_v, acc.at[didx], add=True)
            return carry

        lax.fori_loop(0, n_iter, body, 0)
        plsc.subcore_barrier()
        pltpu.sync_copy(acc.at[pl.ds(r0, RPT)],
                        out_hbm.at[pl.ds(c * NP + r0, RPT)])

    return _deg_kernel


@functools.cache
def _make_agg(F):
    @functools.partial(
        pl.kernel,
        out_type=jax.ShapeDtypeStruct((NSC * NP, F), jnp.float32),
        mesh=_mesh(),
        scratch_types=[
            pltpu.VMEM((CHUNK,), jnp.int32),
            pltpu.VMEM((CHUNK,), jnp.int32),
            pltpu.VMEM((CHUNK, F), jnp.float32),
            pltpu.VMEM_SHARED((NP, F), jnp.float32),
            pltpu.SemaphoreType.DMA,
        ],
    )
    def agg(y_hbm, src_hbm, dst_hbm, zeros_hbm, out_hbm,
            sidx, didx, rows, acc, sem):
        c = lax.axis_index("c")
        s = lax.axis_index("s")
        wid = s * NSC + c
        r0 = s * RPT
        pltpu.sync_copy(zeros_hbm.at[pl.ds(r0, RPT)], acc.at[pl.ds(r0, RPT)])
        plsc.subcore_barrier()

        n_iter = (jnp.int32(NCHUNKS // NW)
                  + (wid < NCHUNKS % NW).astype(jnp.int32))

        def body(i, carry):
            j = wid + NW * i
            pltpu.sync_copy(src_hbm.at[j], sidx)
            pltpu.sync_copy(dst_hbm.at[j], didx)
            pltpu.async_copy(y_hbm.at[sidx], rows, sem).wait()
            pltpu.sync_copy(rows, acc.at[didx], add=True)
            return carry

        lax.fori_loop(0, n_iter, body, 0)
        plsc.subcore_barrier()
        pltpu.sync_copy(acc.at[pl.ds(r0, RPT)],
                        out_hbm.at[pl.ds(c * NP + r0, RPT)])

    return agg


RBLK = 1000


def _deg_d(degp):
    deg = degp[0, :, 0] + degp[1, :, 0] + 1.0
    return lax.rsqrt(deg)


def _t1_body(x_ref, degp_ref, w_ref, o_ref):
    d = _deg_d(degp_ref[...])
    o_ref[...] = jnp.dot(x_ref[...], w_ref[...],
                         preferred_element_type=jnp.float32) * d[:, None]


def _tmid_body(p_ref, y_ref, degp_ref, b_ref, w_ref, o_ref):
    d = _deg_d(degp_ref[...])
    p = p_ref[0] + p_ref[1] + y_ref[...]
    h = jnp.maximum(p * d[:, None] + b_ref[...], 0.0)
    o_ref[...] = jnp.dot(h, w_ref[...],
                         preferred_element_type=jnp.float32) * d[:, None]


def _tout_body(p_ref, y_ref, degp_ref, b_ref, o_ref):
    d = _deg_d(degp_ref[...])
    p = (p_ref[0] + p_ref[1] + y_ref[...])[:, :NCLASS]
    o_ref[...] = p * d[:, None] + b_ref[...]


def _t1(x, degp, W):
    return pl.pallas_call(
        _t1_body,
        grid=(N // RBLK,),
        in_specs=[
            pl.BlockSpec((RBLK, NFEAT), lambda i: (i, 0)),
            pl.BlockSpec((NSC, RBLK, DEG_W), lambda i: (0, i, 0)),
            pl.BlockSpec((NFEAT, NHID), lambda i: (0, 0)),
        ],
        out_specs=pl.BlockSpec((RBLK, NHID), lambda i: (i, 0)),
        out_shape=jax.ShapeDtypeStruct((N, NHID), jnp.float32),
    )(x, degp, W)


def _tmid(p, y, degp, b, W, fout):
    return pl.pallas_call(
        _tmid_body,
        grid=(N // RBLK,),
        in_specs=[
            pl.BlockSpec((NSC, RBLK, NHID), lambda i: (0, i, 0)),
            pl.BlockSpec((RBLK, NHID), lambda i: (i, 0)),
            pl.BlockSpec((NSC, RBLK, DEG_W), lambda i: (0, i, 0)),
            pl.BlockSpec((1, NHID), lambda i: (0, 0)),
            pl.BlockSpec((NHID, fout), lambda i: (0, 0)),
        ],
        out_specs=pl.BlockSpec((RBLK, fout), lambda i: (i, 0)),
        out_shape=jax.ShapeDtypeStruct((N, fout), jnp.float32),
    )(p, y, degp, b, W)


def _tout(p, y, degp, b):
    return pl.pallas_call(
        _tout_body,
        grid=(N // RBLK,),
        in_specs=[
            pl.BlockSpec((NSC, RBLK, NHID), lambda i: (0, i, 0)),
            pl.BlockSpec((RBLK, NHID), lambda i: (i, 0)),
            pl.BlockSpec((NSC, RBLK, DEG_W), lambda i: (0, i, 0)),
            pl.BlockSpec((1, NCLASS), lambda i: (0, 0)),
        ],
        out_specs=pl.BlockSpec((RBLK, NCLASS), lambda i: (i, 0)),
        out_shape=jax.ShapeDtypeStruct((N, NCLASS), jnp.float32),
    )(p, y, degp, b)


def kernel(x, adj, W1, b1, Wx, bx, W2, b2):
    src = adj[0].astype(jnp.int32).reshape(NCHUNKS, CHUNK)
    dst = adj[1].astype(jnp.int32).reshape(NCHUNKS, CHUNK)

    ones8 = jnp.ones((CHUNK, DEG_W), jnp.float32)
    zeros8 = jnp.zeros((NP, DEG_W), jnp.float32)
    zeros128 = jnp.zeros((NP, NHID), jnp.float32)
    W2p = jnp.concatenate([W2, jnp.zeros((NHID, NHID - NCLASS), jnp.float32)],
                          axis=1)

    degp = _deg_kernel_fn()(dst, ones8, zeros8).reshape(NSC, NP, DEG_W)

    y1 = _t1(x, degp, W1)
    p1 = _make_agg(NHID)(y1, src, dst, zeros128).reshape(NSC, NP, NHID)
    y2 = _tmid(p1, y1, degp, b1.reshape(1, NHID), Wx, NHID)
    p2 = _make_agg(NHID)(y2, src, dst, zeros128).reshape(NSC, NP, NHID)
    y3 = _tmid(p2, y2, degp, bx.reshape(1, NHID), W2p, NHID)
    q = _make_agg(NHID)(y3, src, dst, zeros128).reshape(NSC, NP, NHID)
    return _tout(q, y3, degp, b2.reshape(1, NCLASS))
